# baseline (speedup 1.0000x reference)
import functools

import jax
import jax.numpy as jnp
from jax import lax
from jax.experimental import pallas as pl
from jax.experimental.pallas import tpu as pltpu
from jax.experimental.pallas import tpu_sc as plsc

EMB = 16
BATCH = 16384

_info = plsc.get_sparse_core_info()
_NC, _NS = _info.num_cores, _info.num_subcores
_NW = _NC * _NS
_B_PER_W = BATCH // _NW
_WAVE = 16
_N_WAVES = _B_PER_W // _WAVE

_mesh = plsc.VectorSubcoreMesh(core_axis_name="c", subcore_axis_name="s")


@functools.partial(
    pl.kernel,
    mesh=_mesh,
    out_type=jax.ShapeDtypeStruct((EMB, BATCH), jnp.float32),
    compiler_params=pltpu.CompilerParams(needs_layout_passes=False),
    scratch_types=[
        pltpu.VMEM((_B_PER_W,), jnp.int32),
        pltpu.VMEM((3, _WAVE, EMB, 128), jnp.float32),
        pltpu.VMEM((EMB, _B_PER_W), jnp.float32),
        pltpu.SemaphoreType.DMA,
        pltpu.SemaphoreType.DMA,
        pltpu.SemaphoreType.DMA,
    ],
)
def _gather_cols(
    table_hbm, idx_hbm, out_hbm, idx_v, stage_v, cols_v, sem0, sem1, sem2
):
    wid = lax.axis_index("s") * _NC + lax.axis_index("c")
    base = wid * _B_PER_W
    pltpu.sync_copy(idx_hbm.at[pl.ds(base, _B_PER_W)], idx_v)

    lane = lax.iota(jnp.int32, 16)
    sems = (sem0, sem1, sem2)

    def issue(w, buf):
        v = idx_v[pl.ds(w * _WAVE, _WAVE)]
        for u in range(_WAVE):
            c = v[u]
            j = pl.multiple_of((c >> 7) << 7, 128)
            pltpu.async_copy(
                table_hbm.at[:, pl.ds(j, 128)], stage_v.at[buf, u], sems[buf]
            )

    def extract(w, buf):
        v = idx_v[pl.ds(w * _WAVE, _WAVE)]
        pltpu.make_async_copy(
            table_hbm.at[:, pl.ds(0, _WAVE * 128)], stage_v.at[buf], sems[buf]
        ).wait()
        for u in range(_WAVE):
            k = jnp.broadcast_to(v[u] & 127, (16,))
            col = plsc.load_gather(stage_v.at[buf, u], [lane, k])
            plsc.store_scatter(
                cols_v, [lane, jnp.broadcast_to(w * _WAVE + u, (16,))], col
            )

    issue(0, 0)
    issue(1, 1)

    def body(w, carry):
        @pl.when(w % 3 == 0)
        def _():
            issue(w + 2, 2)
            extract(w, 0)

        @pl.when(w % 3 == 1)
        def _():
            issue(w + 2, 0)
            extract(w, 1)

        @pl.when(w % 3 == 2)
        def _():
            issue(w + 2, 1)
            extract(w, 2)

        return carry

    lax.fori_loop(0, _N_WAVES - 2, body, 0)
    extract(_N_WAVES - 2, (_N_WAVES - 2) % 3)
    extract(_N_WAVES - 1, (_N_WAVES - 1) % 3)
    pltpu.sync_copy(cols_v, out_hbm.at[:, pl.ds(base, _B_PER_W)])


def kernel(user_id, table):
    idx = user_id.astype(jnp.int32)
    out_t = _gather_cols(table.T, idx)
    return out_t.T[:, None, :]


# speedup vs baseline: 1.0082x; 1.0082x over previous
import functools

import jax
import jax.numpy as jnp
from jax import lax
from jax.experimental import pallas as pl
from jax.experimental.pallas import tpu as pltpu
from jax.experimental.pallas import tpu_sc as plsc

EMB = 16
BATCH = 16384

_info = plsc.get_sparse_core_info()
_NC, _NS = _info.num_cores, _info.num_subcores
_NW = _NC * _NS
_B_PER_W = BATCH // _NW
_WAVE = 16
_N_WAVES = _B_PER_W // _WAVE

_mesh = plsc.VectorSubcoreMesh(core_axis_name="c", subcore_axis_name="s")


@functools.partial(
    pl.kernel,
    mesh=_mesh,
    out_type=jax.ShapeDtypeStruct((EMB, BATCH), jnp.float32),
    compiler_params=pltpu.CompilerParams(needs_layout_passes=False),
    scratch_types=[
        pltpu.VMEM((_B_PER_W,), jnp.int32),
        pltpu.VMEM((3, _WAVE, EMB, 128), jnp.float32),
        pltpu.VMEM((EMB, _B_PER_W), jnp.float32),
        pltpu.SemaphoreType.DMA,
        pltpu.SemaphoreType.DMA,
        pltpu.SemaphoreType.DMA,
    ],
)
def _gather_cols(
    table_hbm, idx_hbm, out_hbm, idx_v, stage_v, cols_v, sem0, sem1, sem2
):
    wid = lax.axis_index("s") * _NC + lax.axis_index("c")
    base = wid * _B_PER_W
    pltpu.sync_copy(idx_hbm.at[pl.ds(base, _B_PER_W)], idx_v)

    lane = lax.iota(jnp.int32, 16)
    sems = (sem0, sem1, sem2)

    def issue(w, buf):
        v = idx_v[pl.ds(w * _WAVE, _WAVE)]
        for u in range(_WAVE):
            c = v[u]
            j = pl.multiple_of((c >> 7) << 7, 128)
            pltpu.async_copy(
                table_hbm.at[:, pl.ds(j, 128)], stage_v.at[buf, u], sems[buf]
            )

    def extract(w, buf):
        v = idx_v[pl.ds(w * _WAVE, _WAVE)]
        for u in range(_WAVE):
            pltpu.make_async_copy(
                table_hbm.at[:, pl.ds(0, 128)], stage_v.at[buf, u], sems[buf]
            ).wait()
        for u in range(_WAVE):
            k = jnp.broadcast_to(v[u] & 127, (16,))
            col = plsc.load_gather(stage_v.at[buf, u], [lane, k])
            plsc.store_scatter(
                cols_v, [lane, jnp.broadcast_to(w * _WAVE + u, (16,))], col
            )

    issue(0, 0)
    issue(1, 1)

    def body(w, carry):
        @pl.when(w % 3 == 0)
        def _():
            issue(w + 2, 2)
            extract(w, 0)

        @pl.when(w % 3 == 1)
        def _():
            issue(w + 2, 0)
            extract(w, 1)

        @pl.when(w % 3 == 2)
        def _():
            issue(w + 2, 1)
            extract(w, 2)

        return carry

    lax.fori_loop(0, _N_WAVES - 2, body, 0)
    extract(_N_WAVES - 2, (_N_WAVES - 2) % 3)
    extract(_N_WAVES - 1, (_N_WAVES - 1) % 3)
    pltpu.sync_copy(cols_v, out_hbm.at[:, pl.ds(base, _B_PER_W)])


def kernel(user_id, table):
    idx = user_id.astype(jnp.int32)
    out_t = _gather_cols(table.T, idx)
    return out_t.T[:, None, :]
